# baseline (speedup 1.0000x reference)
import functools

import jax
import jax.numpy as jnp
from jax import lax
from jax.experimental import pallas as pl
from jax.experimental.pallas import tpu as pltpu
from jax.experimental.pallas import tpu_sc as plsc

_CHUNK = 128
_NBUF = 5
_K = 2


@functools.lru_cache(maxsize=None)
def _make_gather(total, V, D, NC, NS):
    NW = NC * NS
    assert total % (NW * _CHUNK) == 0
    nstep = total // (NW * _CHUNK)
    b_per_w = nstep * _CHUNK
    mesh = plsc.VectorSubcoreMesh(core_axis_name="c", subcore_axis_name="s")

    @functools.partial(
        pl.kernel,
        out_type=jax.ShapeDtypeStruct((total, D), jnp.float32),
        mesh=mesh,
        scratch_types=[
            pltpu.VMEM((nstep, _CHUNK), jnp.int32),
            pltpu.VMEM_SHARED((V, D), jnp.float32),
            *[pltpu.VMEM((_CHUNK, D), jnp.float32) for _ in range(_NBUF)],
            *[pltpu.SemaphoreType.DMA for _ in range(2 * _NBUF)],
        ],
    )
    def body(table_hbm, idx_hbm, out_hbm, idx_v, table_s, *rest):
        rows = rest[:_NBUF]
        gsems = rest[_NBUF : 2 * _NBUF]
        osems = rest[2 * _NBUF :]
        sid = lax.axis_index("s")
        wid = sid * NC + lax.axis_index("c")
        base = wid * b_per_w
        @pl.when(sid == 0)
        def _():
            pltpu.sync_copy(table_hbm, table_s)

        pltpu.sync_copy(idx_hbm.at[wid], idx_v)
        plsc.subcore_barrier()

        for m in range(_K):
            pltpu.async_copy(table_s.at[idx_v.at[m]], rows[m], gsems[m])

        @pl.loop(0, nstep, step=_NBUF)
        def _(j0):
            for b in range(_NBUF):
                j = j0 + b
                m = j + _K
                bm = (b + _K) % _NBUF

                @pl.when(m < nstep)
                def _():
                    @pl.when(j >= _NBUF - _K)
                    def _():
                        pltpu.make_async_copy(
                            rows[bm],
                            out_hbm.at[pl.ds(base, _CHUNK)],
                            osems[bm],
                        ).wait()

                    pltpu.async_copy(table_s.at[idx_v.at[m]], rows[bm], gsems[bm])

                pltpu.make_async_copy(
                    table_s.at[idx_v.at[b]], rows[b], gsems[b]
                ).wait()
                pltpu.async_copy(
                    rows[b], out_hbm.at[pl.ds(base + j * _CHUNK, _CHUNK)], osems[b]
                )

        for b in range(_NBUF):
            pltpu.make_async_copy(
                rows[b], out_hbm.at[pl.ds(base, _CHUNK)], osems[b]
            ).wait()

    return body


_TC_BLK = 1024
_TC_FRAC_NUM, _TC_FRAC_DEN = 1, 5


@functools.lru_cache(maxsize=None)
def _make_tc_lookup(n_tc, VPAD, D):
    assert n_tc % _TC_BLK == 0

    def body(idx_ref, tab_ref, out_ref):
        iota = lax.broadcasted_iota(jnp.int32, (1, VPAD), 1)
        oh = (idx_ref[...] == iota).astype(jnp.float32)
        out_ref[...] = jnp.dot(oh, tab_ref[...], preferred_element_type=jnp.float32)

    return pl.pallas_call(
        body,
        grid=(n_tc // _TC_BLK,),
        in_specs=[
            pl.BlockSpec((_TC_BLK, 1), lambda i: (i, 0)),
            pl.BlockSpec((VPAD, D), lambda i: (0, 0)),
        ],
        out_specs=pl.BlockSpec((_TC_BLK, D), lambda i: (i, 0)),
        out_shape=jax.ShapeDtypeStruct((n_tc, D), jnp.float32),
    )


def kernel(input, table):
    B, H = input.shape
    V, D = table.shape
    info = plsc.get_sparse_core_info()
    NC, NS = info.num_cores, info.num_subcores
    NW = NC * NS
    total = B * H
    grain = NW * _CHUNK * _NBUF
    n_tc = (total * _TC_FRAC_NUM // _TC_FRAC_DEN) // _TC_BLK * _TC_BLK
    n_sc = total - n_tc
    n_sc -= n_sc % grain
    n_tc = total - n_sc

    flat = input.astype(jnp.int32).reshape(total)
    idx_sc = flat[:n_sc].reshape(NW, n_sc // (NW * _CHUNK), _CHUNK)
    out_sc = _make_gather(n_sc, V, D, NC, NS)(table, idx_sc)

    VPAD = (V + 127) // 128 * 128
    tab_pad = jnp.pad(table, ((0, VPAD - V), (0, 0)))
    idx_tc = flat[n_sc:].reshape(n_tc, 1)
    out_tc = _make_tc_lookup(n_tc, VPAD, D)(idx_tc, tab_pad)

    return jnp.concatenate([out_sc, out_tc], axis=0).reshape(B, H, D)


# speedup vs baseline: 2.9669x; 2.9669x over previous
import functools

import jax
import jax.numpy as jnp
from jax import lax
from jax.experimental import pallas as pl
from jax.experimental.pallas import tpu as pltpu
from jax.experimental.pallas import tpu_sc as plsc

_CHUNK = 128
_NBUF = 5
_K = 1


@functools.lru_cache(maxsize=None)
def _make_gather(total, V, D, NC, NS):
    NW = NC * NS
    assert total % (NW * _CHUNK) == 0
    nstep = total // (NW * _CHUNK)
    b_per_w = nstep * _CHUNK
    mesh = plsc.VectorSubcoreMesh(core_axis_name="c", subcore_axis_name="s")

    @functools.partial(
        pl.kernel,
        out_type=jax.ShapeDtypeStruct((total, D), jnp.float32),
        mesh=mesh,
        scratch_types=[
            pltpu.VMEM((nstep, _CHUNK), jnp.int32),
            pltpu.VMEM_SHARED((V, D), jnp.float32),
            *[pltpu.VMEM((_CHUNK, D), jnp.float32) for _ in range(_NBUF)],
            *[pltpu.SemaphoreType.DMA for _ in range(2 * _NBUF)],
        ],
    )
    def body(table_hbm, idx_hbm, out_hbm, idx_v, table_s, *rest):
        rows = rest[:_NBUF]
        gsems = rest[_NBUF : 2 * _NBUF]
        osems = rest[2 * _NBUF :]
        sid = lax.axis_index("s")
        wid = sid * NC + lax.axis_index("c")
        base = wid * b_per_w
        @pl.when(sid == 0)
        def _():
            pltpu.sync_copy(table_hbm, table_s)

        pltpu.sync_copy(idx_hbm.at[wid], idx_v)
        plsc.subcore_barrier()

        for m in range(_K):
            pltpu.async_copy(table_s.at[idx_v.at[m]], rows[m], gsems[m])

        @pl.loop(0, nstep, step=_NBUF)
        def _(j0):
            for b in range(_NBUF):
                j = j0 + b
                m = j + _K
                bm = (b + _K) % _NBUF

                @pl.when(m < nstep)
                def _():
                    @pl.when(j >= _NBUF - _K)
                    def _():
                        pltpu.make_async_copy(
                            rows[bm],
                            out_hbm.at[pl.ds(base, _CHUNK)],
                            osems[bm],
                        ).wait()

                    pltpu.async_copy(table_s.at[idx_v.at[m]], rows[bm], gsems[bm])

                pltpu.make_async_copy(
                    table_s.at[idx_v.at[b]], rows[b], gsems[b]
                ).wait()
                pltpu.async_copy(
                    rows[b], out_hbm.at[pl.ds(base + j * _CHUNK, _CHUNK)], osems[b]
                )

        for b in range(_NBUF):
            pltpu.make_async_copy(
                rows[b], out_hbm.at[pl.ds(base, _CHUNK)], osems[b]
            ).wait()

    return body


def kernel(input, table):
    B, H = input.shape
    V, D = table.shape
    info = plsc.get_sparse_core_info()
    NC, NS = info.num_cores, info.num_subcores
    total = B * H
    idx = input.astype(jnp.int32).reshape(NC * NS, total // (NC * NS * _CHUNK), _CHUNK)
    out = _make_gather(total, V, D, NC, NS)(table, idx)
    return out.reshape(B, H, D)


# speedup vs baseline: 2.9966x; 1.0100x over previous
import functools

import jax
import jax.numpy as jnp
from jax import lax
from jax.experimental import pallas as pl
from jax.experimental.pallas import tpu as pltpu
from jax.experimental.pallas import tpu_sc as plsc

_CHUNK = 128
_NBUF = 5
_K = 3


@functools.lru_cache(maxsize=None)
def _make_gather(total, V, D, NC, NS):
    NW = NC * NS
    assert total % (NW * _CHUNK) == 0
    nstep = total // (NW * _CHUNK)
    b_per_w = nstep * _CHUNK
    mesh = plsc.VectorSubcoreMesh(core_axis_name="c", subcore_axis_name="s")

    @functools.partial(
        pl.kernel,
        out_type=jax.ShapeDtypeStruct((total, D), jnp.float32),
        mesh=mesh,
        scratch_types=[
            pltpu.VMEM((nstep, _CHUNK), jnp.int32),
            pltpu.VMEM_SHARED((V, D), jnp.float32),
            *[pltpu.VMEM((_CHUNK, D), jnp.float32) for _ in range(_NBUF)],
            *[pltpu.SemaphoreType.DMA for _ in range(2 * _NBUF)],
        ],
    )
    def body(table_hbm, idx_hbm, out_hbm, idx_v, table_s, *rest):
        rows = rest[:_NBUF]
        gsems = rest[_NBUF : 2 * _NBUF]
        osems = rest[2 * _NBUF :]
        sid = lax.axis_index("s")
        wid = sid * NC + lax.axis_index("c")
        base = wid * b_per_w
        @pl.when(sid == 0)
        def _():
            pltpu.sync_copy(table_hbm, table_s)

        pltpu.sync_copy(idx_hbm.at[wid], idx_v)
        plsc.subcore_barrier()

        for m in range(_K):
            pltpu.async_copy(table_s.at[idx_v.at[m]], rows[m], gsems[m])

        @pl.loop(0, nstep, step=_NBUF)
        def _(j0):
            for b in range(_NBUF):
                j = j0 + b
                m = j + _K
                bm = (b + _K) % _NBUF

                @pl.when(m < nstep)
                def _():
                    @pl.when(j >= _NBUF - _K)
                    def _():
                        pltpu.make_async_copy(
                            rows[bm],
                            out_hbm.at[pl.ds(base, _CHUNK)],
                            osems[bm],
                        ).wait()

                    pltpu.async_copy(table_s.at[idx_v.at[m]], rows[bm], gsems[bm])

                pltpu.make_async_copy(
                    table_s.at[idx_v.at[b]], rows[b], gsems[b]
                ).wait()
                pltpu.async_copy(
                    rows[b], out_hbm.at[pl.ds(base + j * _CHUNK, _CHUNK)], osems[b]
                )

        for b in range(_NBUF):
            pltpu.make_async_copy(
                rows[b], out_hbm.at[pl.ds(base, _CHUNK)], osems[b]
            ).wait()

    return body


def kernel(input, table):
    B, H = input.shape
    V, D = table.shape
    info = plsc.get_sparse_core_info()
    NC, NS = info.num_cores, info.num_subcores
    total = B * H
    idx = input.astype(jnp.int32).reshape(NC * NS, total // (NC * NS * _CHUNK), _CHUNK)
    out = _make_gather(total, V, D, NC, NS)(table, idx)
    return out.reshape(B, H, D)


# speedup vs baseline: 2.9981x; 1.0005x over previous
import functools

import jax
import jax.numpy as jnp
from jax import lax
from jax.experimental import pallas as pl
from jax.experimental.pallas import tpu as pltpu
from jax.experimental.pallas import tpu_sc as plsc

_CHUNK = 128
_NBUF = 5
_K = 2


@functools.lru_cache(maxsize=None)
def _make_gather(total, V, D, NC, NS):
    NW = NC * NS
    assert total % (NW * _CHUNK) == 0
    nstep = total // (NW * _CHUNK)
    b_per_w = nstep * _CHUNK
    mesh = plsc.VectorSubcoreMesh(core_axis_name="c", subcore_axis_name="s")

    @functools.partial(
        pl.kernel,
        out_type=jax.ShapeDtypeStruct((total, D), jnp.float32),
        mesh=mesh,
        scratch_types=[
            pltpu.VMEM((nstep, _CHUNK), jnp.int32),
            pltpu.VMEM_SHARED((V, D), jnp.float32),
            *[pltpu.VMEM((_CHUNK, D), jnp.float32) for _ in range(_NBUF)],
            *[pltpu.SemaphoreType.DMA for _ in range(2 * _NBUF)],
        ],
    )
    def body(table_hbm, idx_hbm, out_hbm, idx_v, table_s, *rest):
        rows = rest[:_NBUF]
        gsems = rest[_NBUF : 2 * _NBUF]
        osems = rest[2 * _NBUF :]
        sid = lax.axis_index("s")
        wid = sid * NC + lax.axis_index("c")
        base = wid * b_per_w
        @pl.when(sid == 0)
        def _():
            pltpu.sync_copy(table_hbm, table_s)

        pltpu.sync_copy(idx_hbm.at[wid], idx_v)
        plsc.subcore_barrier()

        for m in range(_K):
            pltpu.async_copy(table_s.at[idx_v.at[m]], rows[m], gsems[m])

        @pl.loop(0, nstep, step=_NBUF)
        def _(j0):
            for b in range(_NBUF):
                j = j0 + b
                m = j + _K
                bm = (b + _K) % _NBUF

                @pl.when(m < nstep)
                def _():
                    @pl.when(j >= _NBUF - _K)
                    def _():
                        pltpu.make_async_copy(
                            rows[bm],
                            out_hbm.at[pl.ds(base, _CHUNK)],
                            osems[bm],
                        ).wait()

                    pltpu.async_copy(table_s.at[idx_v.at[m]], rows[bm], gsems[bm])

                pltpu.make_async_copy(
                    table_s.at[idx_v.at[b]], rows[b], gsems[b]
                ).wait()
                pltpu.async_copy(
                    rows[b], out_hbm.at[pl.ds(base + j * _CHUNK, _CHUNK)], osems[b]
                )

        for b in range(_NBUF):
            pltpu.make_async_copy(
                rows[b], out_hbm.at[pl.ds(base, _CHUNK)], osems[b]
            ).wait()

    return body


def kernel(input, table):
    B, H = input.shape
    V, D = table.shape
    info = plsc.get_sparse_core_info()
    NC, NS = info.num_cores, info.num_subcores
    total = B * H
    idx = input.astype(jnp.int32).reshape(NC * NS, total // (NC * NS * _CHUNK), _CHUNK)
    out = _make_gather(total, V, D, NC, NS)(table, idx)
    return out.reshape(B, H, D)
